# baseline (speedup 1.0000x reference)
import jax
import jax.numpy as jnp
from jax import lax
from jax.experimental import pallas as pl
from jax.experimental.pallas import tpu as pltpu
from jax.experimental.pallas import tpu_sc as plsc

_NUM_CLASSES = 1000000
_FEAT = 64
_BATCH = 16384
_ALPHA = 0.5

_NT = 16
_S = _BATCH // _NT
_SUB = 128
_NSUB = _S // _SUB
_LANES = 16
_FG = _FEAT // _LANES


def _clamp_idx(buf, hi):
    def body(r, _):
        for k in range(_SUB // _LANES):
            sl = pl.ds(k * _LANES, _LANES)
            v = buf[r, sl]
            buf[r, sl] = jnp.minimum(jnp.maximum(v, 0), hi)
        return 0

    lax.fori_loop(0, _NSUB, body, 0)


def _scatter_body(yt2_hbm, ids2_hbm, idxtab_hbm, yt, ids):
    wid = lax.axis_index("s")
    pltpu.sync_copy(yt2_hbm.at[pl.ds(wid * _NSUB, _NSUB)], yt)
    pltpu.sync_copy(ids2_hbm.at[pl.ds(wid * _NSUB, _NSUB)], ids)
    _clamp_idx(yt, _NUM_CLASSES - 1)
    for j in range(_NSUB):
        pltpu.sync_copy(ids.at[j], idxtab_hbm.at[yt.at[j]])


def _loss_body(yt2_hbm, yp_hbm, cen_hbm, idxtab_hbm, zrow_hbm, z1_hbm,
               one1_hbm,
               out_hbm,
               acc, cnt, partials,
               yt, slot, ubuf, gbuf, Dbuf, nbuf, rbuf, ones,
               pbuf, accv, vout):
    wid = lax.axis_index("s")
    base = wid * _S

    pltpu.sync_copy(yt2_hbm.at[pl.ds(wid * _NSUB, _NSUB)], yt)
    _clamp_idx(yt, _NUM_CLASSES - 1)
    pltpu.sync_copy(zrow_hbm, gbuf)
    pltpu.sync_copy(z1_hbm, nbuf)
    pltpu.sync_copy(one1_hbm, ones)
    for j in range(_NSUB):
        pltpu.sync_copy(gbuf, acc.at[pl.ds(base + j * _SUB, _SUB)])
        pltpu.sync_copy(nbuf, cnt.at[pl.ds(base + j * _SUB, _SUB)])
    for j in range(_NSUB):
        pltpu.sync_copy(idxtab_hbm.at[yt.at[j]], slot.at[j])
    _clamp_idx(slot, _BATCH - 1)
    plsc.subcore_barrier()

    for j in range(_NSUB):
        pltpu.sync_copy(yp_hbm.at[pl.ds(base + j * _SUB, _SUB)], ubuf)
        pltpu.sync_copy(cen_hbm.at[yt.at[j]], gbuf)

        def dbody(r, _):
            for c in range(_FG):
                sl = pl.ds(c * _LANES, _LANES)
                Dbuf[r, sl] = ubuf[r, sl] - gbuf[r, sl]
            return 0

        lax.fori_loop(0, _SUB, dbody, 0)
        pltpu.sync_copy(Dbuf, acc.at[slot.at[j]], add=True)
        pltpu.sync_copy(ones, cnt.at[slot.at[j]], add=True)
    plsc.subcore_barrier()

    acc2 = jnp.zeros((_LANES,), jnp.float32)
    for j in range(_NSUB):
        pltpu.sync_copy(yp_hbm.at[pl.ds(base + j * _SUB, _SUB)], ubuf)
        pltpu.sync_copy(cen_hbm.at[yt.at[j]], gbuf)
        pltpu.sync_copy(acc.at[slot.at[j]], Dbuf)
        pltpu.sync_copy(cnt.at[slot.at[j]], nbuf)

        def rbody(k, _):
            sl = pl.ds(k * _LANES, _LANES)
            rbuf[sl] = _ALPHA / (nbuf[sl] + 1.0)
            return 0

        lax.fori_loop(0, _SUB // _LANES, rbody, 0)

        def ebody(r, a):
            s = rbuf[pl.ds(r, _LANES)][0]
            for c in range(_FG):
                sl = pl.ds(c * _LANES, _LANES)
                e = (ubuf[r, sl] - gbuf[r, sl]) - s * Dbuf[r, sl]
                a = a + e * e
            return a

        acc2 = lax.fori_loop(0, _SUB, ebody, acc2)
    accv[...] = acc2
    pltpu.sync_copy(accv, partials.at[wid])
    plsc.subcore_barrier()

    @pl.when(wid == 0)
    def _():
        pltpu.sync_copy(partials, pbuf)
        tot = jnp.zeros((_LANES,), jnp.float32)
        for k in range(_NT):
            tot = tot + pbuf[k, :]
        lane = lax.broadcasted_iota(jnp.int32, (_LANES,), 0)
        for sh in (8, 4, 2, 1):
            perm = (lane + sh) & (_LANES - 1)
            tot = tot + tot.at[perm].get(mode="promise_in_bounds")
        vout[...] = tot * (1.0 / (_BATCH * _FEAT))
        pltpu.sync_copy(vout, out_hbm)


@jax.jit
def kernel(y_true, y_pred, centers):
    yt2 = y_true.astype(jnp.int32).reshape(_BATCH // _SUB, _SUB)
    ids2 = jnp.arange(_BATCH, dtype=jnp.int32).reshape(_BATCH // _SUB, _SUB)
    zrow = jnp.zeros((_SUB, _FEAT), jnp.float32)
    z1 = jnp.zeros((_SUB,), jnp.float32)
    one1 = jnp.ones((_SUB,), jnp.float32)

    mesh = plsc.VectorSubcoreMesh(
        core_axis_name="c", subcore_axis_name="s", num_cores=1)

    scatter_fn = pl.kernel(
        _scatter_body,
        out_type=pltpu.HBM((_NUM_CLASSES,), jnp.int32),
        mesh=mesh,
        compiler_params=pltpu.CompilerParams(use_tc_tiling_on_sc=False),
        scratch_types=[
            pltpu.VMEM((_NSUB, _SUB), jnp.int32),
            pltpu.VMEM((_NSUB, _SUB), jnp.int32),
        ],
    )
    idxtab = scatter_fn(yt2, ids2)

    loss_fn = pl.kernel(
        _loss_body,
        out_type=jax.ShapeDtypeStruct((_LANES,), jnp.float32),
        mesh=mesh,
        compiler_params=pltpu.CompilerParams(use_tc_tiling_on_sc=False),
        scratch_types=[
            pltpu.VMEM_SHARED((_BATCH, _FEAT), jnp.float32),
            pltpu.VMEM_SHARED((_BATCH,), jnp.float32),
            pltpu.VMEM_SHARED((_NT, _LANES), jnp.float32),
            pltpu.VMEM((_NSUB, _SUB), jnp.int32),
            pltpu.VMEM((_NSUB, _SUB), jnp.int32),
            pltpu.VMEM((_SUB, _FEAT), jnp.float32),
            pltpu.VMEM((_SUB, _FEAT), jnp.float32),
            pltpu.VMEM((_SUB, _FEAT), jnp.float32),
            pltpu.VMEM((_SUB,), jnp.float32),
            pltpu.VMEM((_SUB + _LANES,), jnp.float32),
            pltpu.VMEM((_SUB,), jnp.float32),
            pltpu.VMEM((_NT, _LANES), jnp.float32),
            pltpu.VMEM((_LANES,), jnp.float32),
            pltpu.VMEM((_LANES,), jnp.float32),
        ],
    )
    out = loss_fn(yt2, y_pred, centers, idxtab, zrow, z1, one1)
    return out[0]


# speedup vs baseline: 1.0006x; 1.0006x over previous
import jax
import jax.numpy as jnp
from jax import lax
from jax.experimental import pallas as pl
from jax.experimental.pallas import tpu as pltpu
from jax.experimental.pallas import tpu_sc as plsc

_NUM_CLASSES = 1000000
_FEAT = 64
_BATCH = 16384
_ALPHA = 0.5

_NT = 16
_S = _BATCH // _NT
_SUB = 128
_NSUB = _S // _SUB
_LANES = 16
_FG = _FEAT // _LANES


def _clamp_idx(buf, hi):
    def body(r, _):
        for k in range(_SUB // _LANES):
            sl = pl.ds(k * _LANES, _LANES)
            v = buf[r, sl]
            buf[r, sl] = jnp.minimum(jnp.maximum(v, 0), hi)
        return 0

    lax.fori_loop(0, _NSUB, body, 0)


def _scatter_body(yt2_hbm, ids2_hbm, idxtab_hbm, yt, ids):
    wid = lax.axis_index("s")
    pltpu.sync_copy(yt2_hbm.at[pl.ds(wid * _NSUB, _NSUB)], yt)
    pltpu.sync_copy(ids2_hbm.at[pl.ds(wid * _NSUB, _NSUB)], ids)
    _clamp_idx(yt, _NUM_CLASSES - 1)
    for j in range(_NSUB):
        pltpu.sync_copy(ids.at[j], idxtab_hbm.at[yt.at[j]])


def _loss_body(yt2_hbm, yp_hbm, cen_hbm, idxtab_hbm, zrow_hbm, z1_hbm,
               one1_hbm,
               out_hbm,
               acc, cnt, partials,
               yt, slot, ubuf, gbuf, Dbuf, nbuf, rbuf, ones,
               pbuf, accv, vout):
    wid = lax.axis_index("s")
    base = wid * _S

    pltpu.sync_copy(yt2_hbm.at[pl.ds(wid * _NSUB, _NSUB)], yt)
    _clamp_idx(yt, _NUM_CLASSES - 1)
    pltpu.sync_copy(zrow_hbm, Dbuf)
    pltpu.sync_copy(z1_hbm, nbuf)
    pltpu.sync_copy(one1_hbm, ones)
    for j in range(_NSUB):
        pltpu.sync_copy(Dbuf, acc.at[pl.ds(base + j * _SUB, _SUB)])
        pltpu.sync_copy(nbuf, cnt.at[pl.ds(base + j * _SUB, _SUB)])
    for j in range(_NSUB):
        pltpu.sync_copy(idxtab_hbm.at[yt.at[j]], slot.at[j])
    _clamp_idx(slot, _BATCH - 1)
    plsc.subcore_barrier()

    for j in range(_NSUB):
        pltpu.sync_copy(yp_hbm.at[pl.ds(base + j * _SUB, _SUB)], ubuf)
        pltpu.sync_copy(cen_hbm.at[yt.at[j]], gbuf)

        def dbody(r, _):
            for c in range(_FG):
                sl = pl.ds(c * _LANES, _LANES)
                Dbuf[r, sl] = ubuf[r, sl] - gbuf[r, sl]
            return 0

        lax.fori_loop(0, _SUB, dbody, 0)
        pltpu.sync_copy(Dbuf, acc.at[slot.at[j]], add=True)
        pltpu.sync_copy(ones, cnt.at[slot.at[j]], add=True)
    plsc.subcore_barrier()

    acc2 = jnp.zeros((_LANES,), jnp.float32)
    for j in range(_NSUB):
        pltpu.sync_copy(yp_hbm.at[pl.ds(base + j * _SUB, _SUB)], ubuf)
        pltpu.sync_copy(cen_hbm.at[yt.at[j]], gbuf)
        pltpu.sync_copy(acc.at[slot.at[j]], Dbuf)
        pltpu.sync_copy(cnt.at[slot.at[j]], nbuf)

        def rbody(k, _):
            sl = pl.ds(k * _LANES, _LANES)
            rbuf[sl] = _ALPHA / (nbuf[sl] + 1.0)
            return 0

        lax.fori_loop(0, _SUB // _LANES, rbody, 0)

        def ebody(r, a):
            s = rbuf[pl.ds(r, _LANES)][0]
            for c in range(_FG):
                sl = pl.ds(c * _LANES, _LANES)
                e = (ubuf[r, sl] - gbuf[r, sl]) - s * Dbuf[r, sl]
                a = a + e * e
            return a

        acc2 = lax.fori_loop(0, _SUB, ebody, acc2)
    accv[...] = acc2
    pltpu.sync_copy(accv, partials.at[wid])
    plsc.subcore_barrier()

    @pl.when(wid == 0)
    def _():
        pltpu.sync_copy(partials, pbuf)
        tot = jnp.zeros((_LANES,), jnp.float32)
        for k in range(_NT):
            tot = tot + pbuf[k, :]
        lane = lax.broadcasted_iota(jnp.int32, (_LANES,), 0)
        for sh in (8, 4, 2, 1):
            perm = (lane + sh) & (_LANES - 1)
            tot = tot + tot.at[perm].get(mode="promise_in_bounds")
        vout[...] = tot * (1.0 / (_BATCH * _FEAT))
        pltpu.sync_copy(vout, out_hbm)


@jax.jit
def kernel(y_true, y_pred, centers):
    yt2 = y_true.astype(jnp.int32).reshape(_BATCH // _SUB, _SUB)
    ids2 = jnp.arange(_BATCH, dtype=jnp.int32).reshape(_BATCH // _SUB, _SUB)
    zrow = jnp.zeros((_SUB, _FEAT), jnp.float32)
    z1 = jnp.zeros((_SUB,), jnp.float32)
    one1 = jnp.ones((_SUB,), jnp.float32)

    mesh = plsc.VectorSubcoreMesh(
        core_axis_name="c", subcore_axis_name="s", num_cores=1)

    scatter_fn = pl.kernel(
        _scatter_body,
        out_type=pltpu.HBM((_NUM_CLASSES,), jnp.int32),
        mesh=mesh,
        compiler_params=pltpu.CompilerParams(use_tc_tiling_on_sc=False),
        scratch_types=[
            pltpu.VMEM((_NSUB, _SUB), jnp.int32),
            pltpu.VMEM((_NSUB, _SUB), jnp.int32),
        ],
    )
    idxtab = scatter_fn(yt2, ids2)

    loss_fn = pl.kernel(
        _loss_body,
        out_type=jax.ShapeDtypeStruct((_LANES,), jnp.float32),
        mesh=mesh,
        compiler_params=pltpu.CompilerParams(use_tc_tiling_on_sc=False),
        scratch_types=[
            pltpu.VMEM_SHARED((_BATCH, _FEAT), jnp.float32),
            pltpu.VMEM_SHARED((_BATCH,), jnp.float32),
            pltpu.VMEM_SHARED((_NT, _LANES), jnp.float32),
            pltpu.VMEM((_NSUB, _SUB), jnp.int32),
            pltpu.VMEM((_NSUB, _SUB), jnp.int32),
            pltpu.VMEM((_SUB, _FEAT), jnp.float32),
            pltpu.VMEM((_SUB, _FEAT), jnp.float32),
            pltpu.VMEM((_SUB, _FEAT), jnp.float32),
            pltpu.VMEM((_SUB,), jnp.float32),
            pltpu.VMEM((_SUB + _LANES,), jnp.float32),
            pltpu.VMEM((_SUB,), jnp.float32),
            pltpu.VMEM((_NT, _LANES), jnp.float32),
            pltpu.VMEM((_LANES,), jnp.float32),
            pltpu.VMEM((_LANES,), jnp.float32),
        ],
    )
    out = loss_fn(yt2, y_pred, centers, idxtab, zrow, z1, one1)
    return out[0]
